# baseline (speedup 1.0000x reference)
import functools

import jax
import jax.numpy as jnp
from jax import lax
from jax.experimental import pallas as pl
from jax.experimental.pallas import tpu as pltpu
from jax.experimental.pallas import tpu_sc as plsc

_LANES = 16
_IDX_CHUNK = 128
_CW = 512


@functools.cache
def _make_sc_transpose(V, E):
    info = plsc.get_sparse_core_info()
    nc, ns = info.num_cores, info.num_subcores
    nw = nc * ns
    va = (V // _CW) * _CW
    ntail = V - va
    nchunks = va // _CW
    neach = nchunks // nw
    nextra = nchunks - neach * nw
    assert neach % 2 == 1, "pair loop below assumes an odd per-worker count"
    mesh = plsc.VectorSubcoreMesh(core_axis_name="c", subcore_axis_name="s")

    def body(tt_hbm, tail_hbm, out_hbm, in0_v, in1_v, outf_v, tail_v,
             sem0, sem1):
        wid = lax.axis_index("s") * nc + lax.axis_index("c")
        iota = lax.iota(jnp.int32, _LANES)
        iota_e = iota * E

        def cbase(k):
            return (wid + k * nw) * _CW

        def issue(k, buf, sem):
            pltpu.async_copy(tt_hbm.at[:, pl.ds(cbase(k), _CW)], buf, sem)

        def drain(k, buf, sem):
            pltpu.make_async_copy(
                tt_hbm.at[:, pl.ds(cbase(k), _CW)], buf, sem).wait()

        def xpose(k, buf):
            def per_group(g, carry):
                dst = outf_v.at[pl.ds(g * _LANES * E, _LANES * E)]
                for e in range(E):
                    vals = buf[e, pl.ds(g * _LANES, _LANES)]
                    plsc.store_scatter(dst, [iota_e + e], vals)
                return carry

            lax.fori_loop(0, _CW // _LANES, per_group, 0)
            pltpu.sync_copy(outf_v, out_hbm.at[pl.ds(cbase(k) * E, _CW * E)])

        issue(0, in0_v, sem0)

        def pair_step(ii, carry):
            a = 2 * ii
            issue(a + 1, in1_v, sem1)
            drain(a, in0_v, sem0)
            xpose(a, in0_v)
            issue(a + 2, in0_v, sem0)
            drain(a + 1, in1_v, sem1)
            xpose(a + 1, in1_v)
            return carry

        lax.fori_loop(0, neach // 2, pair_step, 0)
        drain(neach - 1, in0_v, sem0)
        xpose(neach - 1, in0_v)

        @pl.when(wid < nextra)
        def _extra():
            base = (nw * neach + wid) * _CW
            pltpu.sync_copy(tt_hbm.at[:, pl.ds(base, _CW)], in1_v)

            def per_group(g, carry):
                dst = outf_v.at[pl.ds(g * _LANES * E, _LANES * E)]
                for e in range(E):
                    vals = in1_v[e, pl.ds(g * _LANES, _LANES)]
                    plsc.store_scatter(dst, [iota_e + e], vals)
                return carry

            lax.fori_loop(0, _CW // _LANES, per_group, 0)
            pltpu.sync_copy(outf_v, out_hbm.at[pl.ds(base * E, _CW * E)])

        if ntail:
            @pl.when(wid == 0)
            def _tail():
                pltpu.sync_copy(tail_hbm, tail_v)
                pltpu.sync_copy(tail_v, out_hbm.at[pl.ds(va * E, ntail * E)])

    return pl.kernel(
        body,
        out_type=jax.ShapeDtypeStruct((V * E,), jnp.float32),
        mesh=mesh,
        compiler_params=pltpu.CompilerParams(
            use_tc_tiling_on_sc=True, needs_layout_passes=False),
        scratch_types=[
            pltpu.VMEM((E, _CW), jnp.float32),
            pltpu.VMEM((E, _CW), jnp.float32),
            pltpu.VMEM((_CW * E,), jnp.float32),
            pltpu.VMEM((max(V - (V // _CW) * _CW, 1) * E,), jnp.float32),
            pltpu.SemaphoreType.DMA,
            pltpu.SemaphoreType.DMA,
        ],
    )


@functools.cache
def _make_pool(B, H, V, E):
    info = plsc.get_sparse_core_info()
    nc, ns = info.num_cores, info.num_subcores
    nw = nc * ns
    bpw = B // nw
    assert B % nw == 0 and E % _LANES == 0
    chunks = [(o, min(_IDX_CHUNK, H - o)) for o in range(0, H, _IDX_CHUNK)]
    mesh = plsc.VectorSubcoreMesh(core_axis_name="c", subcore_axis_name="s")
    ne = E // _LANES
    P = 8

    def body(table_hbm, text_hbm, out_hbm, idx_v, rows0_v, rows1_v,
             pooled_v, sem0, sem1):
        wid = lax.axis_index("s") * nc + lax.axis_index("c")
        base = wid * bpw
        pltpu.sync_copy(text_hbm.at[pl.ds(base, bpw), :], idx_v)

        def issue(i, buf, sem):
            for (o, n) in chunks:
                pltpu.async_copy(
                    table_hbm.at[idx_v.at[i, pl.ds(o, n)]],
                    buf.at[pl.ds(o, n)], sem)

        def drain(i, buf, sem):
            for (o, n) in chunks:
                pltpu.make_async_copy(
                    table_hbm.at[idx_v.at[i, pl.ds(o, n)]],
                    buf.at[pl.ds(o, n)], sem).wait()

        def reduce_into(buf, i):
            def red(jj, accs):
                out = []
                for p in range(P):
                    j = jj * P + p
                    out.append(tuple(
                        accs[p][k] + buf[j, pl.ds(k * _LANES, _LANES)]
                        for k in range(ne)))
                return tuple(out)

            zeros = tuple(
                tuple(jnp.zeros((_LANES,), jnp.float32) for _ in range(ne))
                for _ in range(P))
            accs = lax.fori_loop(0, H // P, red, zeros)
            rem = tuple(accs[0][k] for k in range(ne))
            for p in range(1, P):
                rem = tuple(rem[k] + accs[p][k] for k in range(ne))
            for j in range((H // P) * P, H):
                rem = tuple(rem[k] + buf[j, pl.ds(k * _LANES, _LANES)]
                            for k in range(ne))
            for k in range(ne):
                pooled_v[i, pl.ds(k * _LANES, _LANES)] = rem[k]

        issue(0, rows0_v, sem0)

        def pair_step(ii, carry):
            a = 2 * ii
            issue(a + 1, rows1_v, sem1)
            drain(a, rows0_v, sem0)
            reduce_into(rows0_v, a)
            issue(a + 2, rows0_v, sem0)
            drain(a + 1, rows1_v, sem1)
            reduce_into(rows1_v, a + 1)
            return carry

        lax.fori_loop(0, bpw // 2 - 1, pair_step, 0)
        a = bpw - 2
        issue(a + 1, rows1_v, sem1)
        drain(a, rows0_v, sem0)
        reduce_into(rows0_v, a)
        drain(a + 1, rows1_v, sem1)
        reduce_into(rows1_v, a + 1)

        pltpu.sync_copy(pooled_v, out_hbm.at[pl.ds(base, bpw), :])

    return pl.kernel(
        body,
        out_type=jax.ShapeDtypeStruct((B, E), jnp.float32),
        mesh=mesh,
        compiler_params=pltpu.CompilerParams(use_tc_tiling_on_sc=False),
        scratch_types=[
            pltpu.VMEM((bpw, H), jnp.int32),
            pltpu.VMEM((H, E), jnp.float32),
            pltpu.VMEM((H, E), jnp.float32),
            pltpu.VMEM((bpw, E), jnp.float32),
            pltpu.SemaphoreType.DMA,
            pltpu.SemaphoreType.DMA,
        ],
    )


@functools.cache
def _make_dense(B, H, E, A, O):
    grid = 8
    bt = B // grid

    def body(p_ref, a_ref, w1_ref, w2_ref, b_ref, o_ref):
        t = p_ref[...] * (1.0 / H)
        a = a_ref[...]
        m = jnp.maximum(jnp.max(t, axis=1, keepdims=True),
                        jnp.max(a, axis=1, keepdims=True))
        et = jnp.exp(t - m)
        ea = jnp.exp(a - m)
        s = (jnp.sum(et, axis=1, keepdims=True)
             + jnp.sum(ea, axis=1, keepdims=True))
        acc = jnp.dot(et, w1_ref[...], preferred_element_type=jnp.float32)
        acc = acc + jnp.dot(ea, w2_ref[...], preferred_element_type=jnp.float32)
        o_ref[...] = acc / s + b_ref[...]

    return pl.pallas_call(
        body,
        grid=(grid,),
        in_specs=[
            pl.BlockSpec((bt, E), lambda i: (i, 0)),
            pl.BlockSpec((bt, A), lambda i: (i, 0)),
            pl.BlockSpec((E, O), lambda i: (0, 0)),
            pl.BlockSpec((A, O), lambda i: (0, 0)),
            pl.BlockSpec((1, O), lambda i: (0, 0)),
        ],
        out_specs=pl.BlockSpec((bt, O), lambda i: (i, 0)),
        out_shape=jax.ShapeDtypeStruct((B, O), jnp.float32),
    )


@jax.jit
def kernel(text, audio, table, W, b):
    B, H = text.shape
    V, E = table.shape
    A = audio.shape[1]
    O = W.shape[1]
    va = (V // _CW) * _CW
    tail = table[va:].reshape(-1)
    tflat = _make_sc_transpose(V, E)(table.T, tail)
    pooled = _make_pool(B, H, V, E)(tflat.reshape(V, E), text)
    return _make_dense(B, H, E, A, O)(
        pooled, audio, W[:E], W[E:], b.reshape(1, O))


# speedup vs baseline: 2.4739x; 2.4739x over previous
import functools

import jax
import jax.numpy as jnp
from jax import lax
from jax.experimental import pallas as pl
from jax.experimental.pallas import tpu as pltpu
from jax.experimental.pallas import tpu_sc as plsc

_LANES = 16
_IDX_CHUNK = 128
_COLS = 4096
_SB = 1024
_WIDE = 128


@functools.cache
def _make_transpose(V, E):
    grp = _WIDE // E
    grid = -(-V // _COLS)
    assert _COLS // grp == _SB

    def body(i_ref, o_ref):
        x = i_ref[...]
        z = jnp.concatenate(
            [x[:, q * _SB:(q + 1) * _SB] for q in range(grp)], axis=0)
        o_ref[...] = z.T

    return pl.pallas_call(
        body,
        grid=(grid,),
        in_specs=[pl.BlockSpec((E, _COLS), lambda g: (0, g))],
        out_specs=pl.BlockSpec((_SB, _WIDE), lambda g: (g, 0)),
        out_shape=jax.ShapeDtypeStruct((grid * _SB, _WIDE), jnp.float32),
    )


@functools.cache
def _make_pool(B, H, V, E, vpad):
    info = plsc.get_sparse_core_info()
    nc, ns = info.num_cores, info.num_subcores
    nw = nc * ns
    bpw = B // nw
    grp = _WIDE // E
    assert B % nw == 0 and E % _LANES == 0
    chunks = [(o, min(_IDX_CHUNK, H - o)) for o in range(0, H, _IDX_CHUNK)]
    goffs = list(range(0, H - _LANES + 1, _LANES))
    if H % _LANES:
        goffs.append(H - _LANES)
    mesh = plsc.VectorSubcoreMesh(core_axis_name="c", subcore_axis_name="s")
    ne = E // _LANES
    P = 8
    cols_sh = _COLS.bit_length() - 1
    sb_sh = _SB.bit_length() - 1
    grp_sh = grp.bit_length() - 1

    def body(table_hbm, text_hbm, out_hbm, txt_v, idx_v, rows0_v, rows1_v,
             pooled_v, sem0, sem1):
        wid = lax.axis_index("s") * nc + lax.axis_index("c")
        base = wid * bpw
        pltpu.sync_copy(text_hbm.at[pl.ds(base, bpw), :], txt_v)

        def xform(i, carry):
            for o in goffs:
                v = txt_v[i, pl.ds(o, _LANES)]
                a = (v >> cols_sh) << sb_sh
                bb = v & (_SB - 1)
                c = (v >> sb_sh) & (grp - 1)
                idx_v[i, pl.ds(o, _LANES)] = ((a + bb) << grp_sh) + c
            return carry

        lax.fori_loop(0, bpw, xform, 0)

        def issue(i, buf, sem):
            for (o, n) in chunks:
                pltpu.async_copy(
                    table_hbm.at[idx_v.at[i, pl.ds(o, n)]],
                    buf.at[pl.ds(o, n)], sem)

        def drain(i, buf, sem):
            for (o, n) in chunks:
                pltpu.make_async_copy(
                    table_hbm.at[idx_v.at[i, pl.ds(o, n)]],
                    buf.at[pl.ds(o, n)], sem).wait()

        def reduce_into(buf, i):
            def red(jj, accs):
                out = []
                for p in range(P):
                    j = jj * P + p
                    out.append(tuple(
                        accs[p][k] + buf[j, pl.ds(k * _LANES, _LANES)]
                        for k in range(ne)))
                return tuple(out)

            zeros = tuple(
                tuple(jnp.zeros((_LANES,), jnp.float32) for _ in range(ne))
                for _ in range(P))
            accs = lax.fori_loop(0, H // P, red, zeros)
            rem = tuple(accs[0][k] for k in range(ne))
            for p in range(1, P):
                rem = tuple(rem[k] + accs[p][k] for k in range(ne))
            for j in range((H // P) * P, H):
                rem = tuple(rem[k] + buf[j, pl.ds(k * _LANES, _LANES)]
                            for k in range(ne))
            for k in range(ne):
                pooled_v[i, pl.ds(k * _LANES, _LANES)] = rem[k]

        issue(0, rows0_v, sem0)

        def pair_step(ii, carry):
            a = 2 * ii
            issue(a + 1, rows1_v, sem1)
            drain(a, rows0_v, sem0)
            reduce_into(rows0_v, a)
            issue(a + 2, rows0_v, sem0)
            drain(a + 1, rows1_v, sem1)
            reduce_into(rows1_v, a + 1)
            return carry

        lax.fori_loop(0, bpw // 2 - 1, pair_step, 0)
        a = bpw - 2
        issue(a + 1, rows1_v, sem1)
        drain(a, rows0_v, sem0)
        reduce_into(rows0_v, a)
        drain(a + 1, rows1_v, sem1)
        reduce_into(rows1_v, a + 1)

        pltpu.sync_copy(pooled_v, out_hbm.at[pl.ds(base, bpw), :])

    return pl.kernel(
        body,
        out_type=jax.ShapeDtypeStruct((B, E), jnp.float32),
        mesh=mesh,
        compiler_params=pltpu.CompilerParams(use_tc_tiling_on_sc=False),
        scratch_types=[
            pltpu.VMEM((bpw, H), jnp.int32),
            pltpu.VMEM((bpw, H), jnp.int32),
            pltpu.VMEM((H, E), jnp.float32),
            pltpu.VMEM((H, E), jnp.float32),
            pltpu.VMEM((bpw, E), jnp.float32),
            pltpu.SemaphoreType.DMA,
            pltpu.SemaphoreType.DMA,
        ],
    )


@functools.cache
def _make_dense(B, H, E, A, O):
    grid = 8
    bt = B // grid

    def body(p_ref, a_ref, w1_ref, w2_ref, b_ref, o_ref):
        t = p_ref[...] * (1.0 / H)
        a = a_ref[...]
        m = jnp.maximum(jnp.max(t, axis=1, keepdims=True),
                        jnp.max(a, axis=1, keepdims=True))
        et = jnp.exp(t - m)
        ea = jnp.exp(a - m)
        s = (jnp.sum(et, axis=1, keepdims=True)
             + jnp.sum(ea, axis=1, keepdims=True))
        acc = jnp.dot(et, w1_ref[...], preferred_element_type=jnp.float32)
        acc = acc + jnp.dot(ea, w2_ref[...], preferred_element_type=jnp.float32)
        o_ref[...] = acc / s + b_ref[...]

    return pl.pallas_call(
        body,
        grid=(grid,),
        in_specs=[
            pl.BlockSpec((bt, E), lambda i: (i, 0)),
            pl.BlockSpec((bt, A), lambda i: (i, 0)),
            pl.BlockSpec((E, O), lambda i: (0, 0)),
            pl.BlockSpec((A, O), lambda i: (0, 0)),
            pl.BlockSpec((1, O), lambda i: (0, 0)),
        ],
        out_specs=pl.BlockSpec((bt, O), lambda i: (i, 0)),
        out_shape=jax.ShapeDtypeStruct((B, O), jnp.float32),
    )


@jax.jit
def kernel(text, audio, table, W, b):
    B, H = text.shape
    V, E = table.shape
    A = audio.shape[1]
    O = W.shape[1]
    tablerm = _make_transpose(V, E)(table.T)
    vpad = tablerm.shape[0] * (_WIDE // E)
    pooled = _make_pool(B, H, V, E, vpad)(tablerm.reshape(vpad, E), text)
    return _make_dense(B, H, E, A, O)(
        pooled, audio, W[:E], W[E:], b.reshape(1, O))


# speedup vs baseline: 3.0387x; 1.2283x over previous
import functools

import jax
import jax.numpy as jnp
from jax import lax
from jax.experimental import pallas as pl
from jax.experimental.pallas import tpu as pltpu
from jax.experimental.pallas import tpu_sc as plsc

_LANES = 16
_IDX_CHUNK = 128
_COLS = 8192
_SB = 2048
_WIDE = 128


@functools.cache
def _make_transpose(V, E):
    grp = _WIDE // E
    grid = -(-V // _COLS)
    assert _COLS // grp == _SB

    def body(i_ref, o_ref):
        x = i_ref[...]
        z = jnp.concatenate(
            [x[:, q * _SB:(q + 1) * _SB] for q in range(grp)], axis=0)
        o_ref[...] = z.T

    return pl.pallas_call(
        body,
        grid=(grid,),
        in_specs=[pl.BlockSpec((E, _COLS), lambda g: (0, g))],
        out_specs=pl.BlockSpec((_SB, _WIDE), lambda g: (g, 0)),
        out_shape=jax.ShapeDtypeStruct((grid * _SB, _WIDE), jnp.float32),
    )


@functools.cache
def _make_pool(B, H, V, E, vpad):
    info = plsc.get_sparse_core_info()
    nc, ns = info.num_cores, info.num_subcores
    nw = nc * ns
    bpw = B // nw
    grp = _WIDE // E
    assert B % nw == 0 and E % _LANES == 0
    chunks = [(o, min(_IDX_CHUNK, H - o)) for o in range(0, H, _IDX_CHUNK)]
    goffs = list(range(0, H - _LANES + 1, _LANES))
    if H % _LANES:
        goffs.append(H - _LANES)
    mesh = plsc.VectorSubcoreMesh(core_axis_name="c", subcore_axis_name="s")
    ne = E // _LANES
    P = 8
    cols_sh = _COLS.bit_length() - 1
    sb_sh = _SB.bit_length() - 1
    grp_sh = grp.bit_length() - 1

    def body(table_hbm, text_hbm, out_hbm, txt_v, idx_v, rows0_v, rows1_v,
             pooled_v, sem0, sem1):
        wid = lax.axis_index("s") * nc + lax.axis_index("c")
        base = wid * bpw
        pltpu.sync_copy(text_hbm.at[pl.ds(base, bpw), :], txt_v)

        def xform(i, carry):
            for o in goffs:
                v = txt_v[i, pl.ds(o, _LANES)]
                a = (v >> cols_sh) << sb_sh
                bb = v & (_SB - 1)
                c = (v >> sb_sh) & (grp - 1)
                idx_v[i, pl.ds(o, _LANES)] = ((a + bb) << grp_sh) + c
            return carry

        lax.fori_loop(0, bpw, xform, 0)

        def issue(i, buf, sem):
            for (o, n) in chunks:
                pltpu.async_copy(
                    table_hbm.at[idx_v.at[i, pl.ds(o, n)]],
                    buf.at[pl.ds(o, n)], sem)

        def drain(i, buf, sem):
            for (o, n) in chunks:
                pltpu.make_async_copy(
                    table_hbm.at[idx_v.at[i, pl.ds(o, n)]],
                    buf.at[pl.ds(o, n)], sem).wait()

        def reduce_into(buf, i):
            def red(jj, accs):
                out = []
                for p in range(P):
                    j = jj * P + p
                    out.append(tuple(
                        accs[p][k] + buf[j, pl.ds(k * _LANES, _LANES)]
                        for k in range(ne)))
                return tuple(out)

            zeros = tuple(
                tuple(jnp.zeros((_LANES,), jnp.float32) for _ in range(ne))
                for _ in range(P))
            accs = lax.fori_loop(0, H // P, red, zeros)
            rem = tuple(accs[0][k] for k in range(ne))
            for p in range(1, P):
                rem = tuple(rem[k] + accs[p][k] for k in range(ne))
            for j in range((H // P) * P, H):
                rem = tuple(rem[k] + buf[j, pl.ds(k * _LANES, _LANES)]
                            for k in range(ne))
            for k in range(ne):
                pooled_v[i, pl.ds(k * _LANES, _LANES)] = rem[k]

        issue(0, rows0_v, sem0)

        def pair_step(ii, carry):
            a = 2 * ii
            issue(a + 1, rows1_v, sem1)
            drain(a, rows0_v, sem0)
            reduce_into(rows0_v, a)
            issue(a + 2, rows0_v, sem0)
            drain(a + 1, rows1_v, sem1)
            reduce_into(rows1_v, a + 1)
            return carry

        lax.fori_loop(0, bpw // 2 - 1, pair_step, 0)
        a = bpw - 2
        issue(a + 1, rows1_v, sem1)
        drain(a, rows0_v, sem0)
        reduce_into(rows0_v, a)
        drain(a + 1, rows1_v, sem1)
        reduce_into(rows1_v, a + 1)

        pltpu.sync_copy(pooled_v, out_hbm.at[pl.ds(base, bpw), :])

    return pl.kernel(
        body,
        out_type=jax.ShapeDtypeStruct((B, E), jnp.float32),
        mesh=mesh,
        compiler_params=pltpu.CompilerParams(use_tc_tiling_on_sc=False),
        scratch_types=[
            pltpu.VMEM((bpw, H), jnp.int32),
            pltpu.VMEM((bpw, H), jnp.int32),
            pltpu.VMEM((H, E), jnp.float32),
            pltpu.VMEM((H, E), jnp.float32),
            pltpu.VMEM((bpw, E), jnp.float32),
            pltpu.SemaphoreType.DMA,
            pltpu.SemaphoreType.DMA,
        ],
    )


@functools.cache
def _make_dense(B, H, E, A, O):
    grid = 8
    bt = B // grid

    def body(p_ref, a_ref, w1_ref, w2_ref, b_ref, o_ref):
        t = p_ref[...] * (1.0 / H)
        a = a_ref[...]
        m = jnp.maximum(jnp.max(t, axis=1, keepdims=True),
                        jnp.max(a, axis=1, keepdims=True))
        et = jnp.exp(t - m)
        ea = jnp.exp(a - m)
        s = (jnp.sum(et, axis=1, keepdims=True)
             + jnp.sum(ea, axis=1, keepdims=True))
        acc = jnp.dot(et, w1_ref[...], preferred_element_type=jnp.float32)
        acc = acc + jnp.dot(ea, w2_ref[...], preferred_element_type=jnp.float32)
        o_ref[...] = acc / s + b_ref[...]

    return pl.pallas_call(
        body,
        grid=(grid,),
        in_specs=[
            pl.BlockSpec((bt, E), lambda i: (i, 0)),
            pl.BlockSpec((bt, A), lambda i: (i, 0)),
            pl.BlockSpec((E, O), lambda i: (0, 0)),
            pl.BlockSpec((A, O), lambda i: (0, 0)),
            pl.BlockSpec((1, O), lambda i: (0, 0)),
        ],
        out_specs=pl.BlockSpec((bt, O), lambda i: (i, 0)),
        out_shape=jax.ShapeDtypeStruct((B, O), jnp.float32),
    )


@jax.jit
def kernel(text, audio, table, W, b):
    B, H = text.shape
    V, E = table.shape
    A = audio.shape[1]
    O = W.shape[1]
    tablerm = _make_transpose(V, E)(table.T)
    vpad = tablerm.shape[0] * (_WIDE // E)
    pooled = _make_pool(B, H, V, E, vpad)(tablerm.reshape(vpad, E), text)
    return _make_dense(B, H, E, A, O)(
        pooled, audio, W[:E], W[E:], b.reshape(1, O))


# speedup vs baseline: 3.5366x; 1.1639x over previous
import functools

import jax
import jax.numpy as jnp
from jax import lax
from jax.experimental import pallas as pl
from jax.experimental.pallas import tpu as pltpu
from jax.experimental.pallas import tpu_sc as plsc

_LANES = 16
_IDX_CHUNK = 128
_COLS = 16384
_SB = 4096
_WIDE = 128


@functools.cache
def _make_transpose(V, E):
    grp = _WIDE // E
    grid = -(-V // _COLS)
    assert _COLS // grp == _SB

    def body(i_ref, o_ref):
        x = i_ref[...]
        z = jnp.concatenate(
            [x[:, q * _SB:(q + 1) * _SB] for q in range(grp)], axis=0)
        o_ref[...] = z.T

    return pl.pallas_call(
        body,
        grid=(grid,),
        in_specs=[pl.BlockSpec((E, _COLS), lambda g: (0, g))],
        out_specs=pl.BlockSpec((_SB, _WIDE), lambda g: (g, 0)),
        out_shape=jax.ShapeDtypeStruct((grid * _SB, _WIDE), jnp.float32),
    )


@functools.cache
def _make_pool(B, H, V, E, vpad):
    info = plsc.get_sparse_core_info()
    nc, ns = info.num_cores, info.num_subcores
    nw = nc * ns
    bpw = B // nw
    grp = _WIDE // E
    assert B % nw == 0 and E % _LANES == 0
    chunks = [(o, min(_IDX_CHUNK, H - o)) for o in range(0, H, _IDX_CHUNK)]
    goffs = list(range(0, H - _LANES + 1, _LANES))
    if H % _LANES:
        goffs.append(H - _LANES)
    mesh = plsc.VectorSubcoreMesh(core_axis_name="c", subcore_axis_name="s")
    ne = E // _LANES
    P = 8
    cols_sh = _COLS.bit_length() - 1
    sb_sh = _SB.bit_length() - 1
    grp_sh = grp.bit_length() - 1

    def body(table_hbm, text_hbm, out_hbm, txt_v, idx_v, rows0_v, rows1_v,
             pooled_v, sem0, sem1):
        wid = lax.axis_index("s") * nc + lax.axis_index("c")
        base = wid * bpw
        pltpu.sync_copy(text_hbm.at[pl.ds(base, bpw), :], txt_v)

        def xform(i, carry):
            for o in goffs:
                v = txt_v[i, pl.ds(o, _LANES)]
                a = (v >> cols_sh) << sb_sh
                bb = v & (_SB - 1)
                c = (v >> sb_sh) & (grp - 1)
                idx_v[i, pl.ds(o, _LANES)] = ((a + bb) << grp_sh) + c
            return carry

        lax.fori_loop(0, bpw, xform, 0)

        def issue(i, buf, sem):
            for (o, n) in chunks:
                pltpu.async_copy(
                    table_hbm.at[idx_v.at[i, pl.ds(o, n)]],
                    buf.at[pl.ds(o, n)], sem)

        def drain(i, buf, sem):
            for (o, n) in chunks:
                pltpu.make_async_copy(
                    table_hbm.at[idx_v.at[i, pl.ds(o, n)]],
                    buf.at[pl.ds(o, n)], sem).wait()

        def reduce_into(buf, i):
            def red(jj, accs):
                out = []
                for p in range(P):
                    j = jj * P + p
                    out.append(tuple(
                        accs[p][k] + buf[j, pl.ds(k * _LANES, _LANES)]
                        for k in range(ne)))
                return tuple(out)

            zeros = tuple(
                tuple(jnp.zeros((_LANES,), jnp.float32) for _ in range(ne))
                for _ in range(P))
            accs = lax.fori_loop(0, H // P, red, zeros)
            rem = tuple(accs[0][k] for k in range(ne))
            for p in range(1, P):
                rem = tuple(rem[k] + accs[p][k] for k in range(ne))
            for j in range((H // P) * P, H):
                rem = tuple(rem[k] + buf[j, pl.ds(k * _LANES, _LANES)]
                            for k in range(ne))
            for k in range(ne):
                pooled_v[i, pl.ds(k * _LANES, _LANES)] = rem[k]

        issue(0, rows0_v, sem0)

        def pair_step(ii, carry):
            a = 2 * ii
            issue(a + 1, rows1_v, sem1)
            drain(a, rows0_v, sem0)
            reduce_into(rows0_v, a)
            issue(a + 2, rows0_v, sem0)
            drain(a + 1, rows1_v, sem1)
            reduce_into(rows1_v, a + 1)
            return carry

        lax.fori_loop(0, bpw // 2 - 1, pair_step, 0)
        a = bpw - 2
        issue(a + 1, rows1_v, sem1)
        drain(a, rows0_v, sem0)
        reduce_into(rows0_v, a)
        drain(a + 1, rows1_v, sem1)
        reduce_into(rows1_v, a + 1)

        pltpu.sync_copy(pooled_v, out_hbm.at[pl.ds(base, bpw), :])

    return pl.kernel(
        body,
        out_type=jax.ShapeDtypeStruct((B, E), jnp.float32),
        mesh=mesh,
        compiler_params=pltpu.CompilerParams(use_tc_tiling_on_sc=False),
        scratch_types=[
            pltpu.VMEM((bpw, H), jnp.int32),
            pltpu.VMEM((bpw, H), jnp.int32),
            pltpu.VMEM((H, E), jnp.float32),
            pltpu.VMEM((H, E), jnp.float32),
            pltpu.VMEM((bpw, E), jnp.float32),
            pltpu.SemaphoreType.DMA,
            pltpu.SemaphoreType.DMA,
        ],
    )


@functools.cache
def _make_dense(B, H, E, A, O):
    grid = 8
    bt = B // grid

    def body(p_ref, a_ref, w1_ref, w2_ref, b_ref, o_ref):
        t = p_ref[...] * (1.0 / H)
        a = a_ref[...]
        m = jnp.maximum(jnp.max(t, axis=1, keepdims=True),
                        jnp.max(a, axis=1, keepdims=True))
        et = jnp.exp(t - m)
        ea = jnp.exp(a - m)
        s = (jnp.sum(et, axis=1, keepdims=True)
             + jnp.sum(ea, axis=1, keepdims=True))
        acc = jnp.dot(et, w1_ref[...], preferred_element_type=jnp.float32)
        acc = acc + jnp.dot(ea, w2_ref[...], preferred_element_type=jnp.float32)
        o_ref[...] = acc / s + b_ref[...]

    return pl.pallas_call(
        body,
        grid=(grid,),
        in_specs=[
            pl.BlockSpec((bt, E), lambda i: (i, 0)),
            pl.BlockSpec((bt, A), lambda i: (i, 0)),
            pl.BlockSpec((E, O), lambda i: (0, 0)),
            pl.BlockSpec((A, O), lambda i: (0, 0)),
            pl.BlockSpec((1, O), lambda i: (0, 0)),
        ],
        out_specs=pl.BlockSpec((bt, O), lambda i: (i, 0)),
        out_shape=jax.ShapeDtypeStruct((B, O), jnp.float32),
    )


@jax.jit
def kernel(text, audio, table, W, b):
    B, H = text.shape
    V, E = table.shape
    A = audio.shape[1]
    O = W.shape[1]
    tablerm = _make_transpose(V, E)(table.T)
    vpad = tablerm.shape[0] * (_WIDE // E)
    pooled = _make_pool(B, H, V, E, vpad)(tablerm.reshape(vpad, E), text)
    return _make_dense(B, H, E, A, O)(
        pooled, audio, W[:E], W[E:], b.reshape(1, O))
